# baseline (speedup 1.0000x reference)
import functools

import jax
import jax.numpy as jnp
from jax import lax
from jax.experimental import pallas as pl
from jax.experimental.pallas import tpu as pltpu
from jax.experimental.pallas import tpu_sc as plsc

_SIZE = 512
_L = 16
_NC = 2
_NS = 16


@functools.lru_cache(maxsize=None)
def _make_project(T, P, NV, NU):
    tpad = -(-T // _L) * _L
    nchunk = P // _L

    mesh = plsc.VectorSubcoreMesh(
        core_axis_name="c", subcore_axis_name="s", num_cores=_NC, num_subcores=_NS
    )

    @functools.partial(
        pl.kernel,
        out_type=jax.ShapeDtypeStruct((P, 3), jnp.float32),
        mesh=mesh,
        compiler_params=pltpu.CompilerParams(
            needs_layout_passes=False, use_tc_tiling_on_sc=True
        ),
        scratch_types=[
            pltpu.VMEM((NV, 3), jnp.float32),
            pltpu.VMEM((NU, 2), jnp.float32),
            pltpu.VMEM((T, 3), jnp.int32),
            pltpu.VMEM((T, 3), jnp.int32),
            pltpu.VMEM((_L, 2), jnp.float32),
            pltpu.VMEM((tpad * 32,), jnp.float32),
            pltpu.VMEM((_L, 3), jnp.float32),
            pltpu.SemaphoreType.DMA,
            pltpu.SemaphoreType.DMA,
            pltpu.SemaphoreType.DMA,
            pltpu.SemaphoreType.DMA,
            pltpu.SemaphoreType.DMA,
        ],
    )
    def project(vert_hbm, uv_hbm, fac_hbm, ufac_hbm, pts_hbm, out_hbm,
                vertv, uvv, facv, ufacv, ptsv, tab, obuf, s0, s1, s2, s3, s4):
        wid = lax.axis_index("s") * _NC + lax.axis_index("c")

        @pl.when(wid < nchunk)
        def _():
            d0 = pltpu.async_copy(vert_hbm, vertv, s0)
            d1 = pltpu.async_copy(uv_hbm, uvv, s1)
            d2 = pltpu.async_copy(fac_hbm, facv, s2)
            d3 = pltpu.async_copy(ufac_hbm, ufacv, s3)
            d4 = pltpu.async_copy(pts_hbm.at[pl.ds(wid * _L, _L)], ptsv, s4)
            d0.wait()
            d1.wait()
            d2.wait()
            d3.wait()
            d4.wait()

            iota = lax.broadcasted_iota(jnp.int32, (_L,), 0)
            c0 = jnp.zeros((_L,), jnp.int32)
            c1 = c0 + 1
            c2 = c0 + 2

            for g in range(tpad // _L):
                lt = iota + (g * _L)
                if (g + 1) * _L > T:
                    lt = jnp.minimum(lt, T - 1)
                fi0 = plsc.load_gather(facv, [lt, c0])
                fi1 = plsc.load_gather(facv, [lt, c1])
                fi2 = plsc.load_gather(facv, [lt, c2])
                uf0 = plsc.load_gather(ufacv, [lt, c0])
                uf1 = plsc.load_gather(ufacv, [lt, c1])
                uf2 = plsc.load_gather(ufacv, [lt, c2])

                ax = plsc.load_gather(vertv, [fi0, c0])
                ay = plsc.load_gather(vertv, [fi0, c1])
                az = plsc.load_gather(vertv, [fi0, c2])
                bx = plsc.load_gather(vertv, [fi1, c0])
                by = plsc.load_gather(vertv, [fi1, c1])
                bz = plsc.load_gather(vertv, [fi1, c2])
                cx = plsc.load_gather(vertv, [fi2, c0])
                cy = plsc.load_gather(vertv, [fi2, c1])
                cz = plsc.load_gather(vertv, [fi2, c2])
                ua = plsc.load_gather(uvv, [uf0, c0])
                va = plsc.load_gather(uvv, [uf0, c1])
                ub = plsc.load_gather(uvv, [uf1, c0])
                vb = plsc.load_gather(uvv, [uf1, c1])
                uc = plsc.load_gather(uvv, [uf2, c0])
                vc = plsc.load_gather(uvv, [uf2, c1])

                cross = (bx - ax) * (cy - ay) - (by - ay) * (cx - ax)
                w = 0.5 * cross
                valid = (cross > 0.0) & (w >= 1e-9)
                wsafe = jnp.where(w == 0.0, 1.0, w)
                h = 0.5 / wsafe

                def edge(qx, qy, rx, ry):
                    return ((qx * ry - qy * rx) * h,
                            (qy - ry) * h,
                            (rx - qx) * h)

                w1c0, w1cx, w1cy = edge(bx, by, cx, cy)
                w2c0, w2cx, w2cy = edge(cx, cy, ax, ay)
                a0c0, a0cx, a0cy = edge(ax, ay, bx, by)

                inf = jnp.float32(jnp.inf)
                xmin = jnp.where(valid, jnp.minimum(jnp.minimum(ax, bx), cx), inf)
                xmax = jnp.where(valid, jnp.maximum(jnp.maximum(ax, bx), cx), -inf)
                ymin = jnp.minimum(jnp.minimum(ay, by), cy)
                ymax = jnp.maximum(jnp.maximum(ay, by), cy)

                zia = 1.0 / az
                zib = 1.0 / bz
                zic = 1.0 / cz
                rows = [
                    xmin, xmax, ymin, ymax,
                    w1c0, w1cx, w1cy,
                    w2c0, w2cx, w2cy,
                    a0c0, a0cx, a0cy,
                    ua * zia, ub * zib, uc * zic,
                    va * zia, vb * zib, vc * zic,
                    zia, zib, zic,
                ]
                lanes = iota + (g * _L)
                for k, val in enumerate(rows):
                    plsc.store_scatter(tab, [lanes * 32 + k], val)

            px = plsc.load_gather(ptsv, [iota, c0])
            py = plsc.load_gather(ptsv, [iota, c1])
            px = px / (_SIZE - 1) * 2.0 - 1.0
            py = (_SIZE - py) / (_SIZE - 1) * 2.0 - 1.0

            bs = jnp.full((_L,), -jnp.inf, jnp.float32)
            bu = jnp.zeros((_L,), jnp.float32)
            bv = jnp.zeros((_L,), jnp.float32)
            for t in range(T):
                ca = tab[pl.ds(t * 32, _L)]
                cb = tab[pl.ds(t * 32 + _L, _L)]
                inb = ((px >= ca[0]) & (px <= ca[1])
                       & (py >= ca[2]) & (py <= ca[3]))
                w1 = ca[4] + ca[5] * px + ca[6] * py
                w2 = ca[7] + ca[8] * px + ca[9] * py
                a0 = ca[10] + ca[11] * px + ca[12] * py
                w3 = 1.0 - w1 - w2
                zi = w1 * cb[3] + w2 * cb[4] + w3 * cb[5]
                ptz = 1.0 / zi
                uu = (w1 * ca[13] + w2 * ca[14] + w3 * ca[15]) * ptz
                vv = (w1 * cb[0] + w2 * cb[1] + w3 * cb[2]) * ptz
                upd = (inb & (w1 >= 0.0) & (w2 >= 0.0) & (a0 >= 0.0)
                       & (ptz > bs))
                bs = jnp.where(upd, ptz, bs)
                bu = jnp.where(upd, uu, bu)
                bv = jnp.where(upd, vv, bv)

            plsc.store_scatter(obuf, [iota, c0], bu)
            plsc.store_scatter(obuf, [iota, c1], bv)
            plsc.store_scatter(obuf, [iota, c2], bs)
            pltpu.sync_copy(obuf, out_hbm.at[pl.ds(wid * _L, _L)])

    return project


def kernel(vertices, points, faces, uv, uvfaces):
    T = faces.shape[0]
    P = points.shape[0]
    NV = vertices.shape[0]
    NU = uv.shape[0]

    return _make_project(T, P, NV, NU)(
        vertices,
        uv,
        faces.astype(jnp.int32),
        uvfaces.astype(jnp.int32),
        points,
    )


# speedup vs baseline: 1.0600x; 1.0600x over previous
import functools

import jax
import jax.numpy as jnp
from jax import lax
from jax.experimental import pallas as pl
from jax.experimental.pallas import tpu as pltpu
from jax.experimental.pallas import tpu_sc as plsc

_SIZE = 512
_L = 16
_NC = 2
_NS = 16


@functools.lru_cache(maxsize=None)
def _make_project(T, P, NV, NU):
    tpad = -(-T // _L) * _L
    nchunk = P // _L
    OV = 0
    OU = 3 * NV
    OF = OU + 2 * NU
    OG = OF + 3 * T
    OP = OG + 3 * T
    NIN = OP + 2 * P

    mesh = plsc.VectorSubcoreMesh(
        core_axis_name="c", subcore_axis_name="s", num_cores=_NC, num_subcores=_NS
    )

    @functools.partial(
        pl.kernel,
        out_type=jax.ShapeDtypeStruct((P * 3,), jnp.float32),
        mesh=mesh,
        compiler_params=pltpu.CompilerParams(
            needs_layout_passes=False, skip_device_barrier=True
        ),
        scratch_types=[
            pltpu.VMEM((NIN,), jnp.float32),
            pltpu.VMEM((tpad * 32,), jnp.float32),
            pltpu.VMEM((3 * _L,), jnp.float32),
            pltpu.SemaphoreType.DMA,
        ],
    )
    def project(in_hbm, out_hbm, buf, tab, obuf, s0):
        wid = lax.axis_index("s") * _NC + lax.axis_index("c")

        @pl.when(wid < nchunk)
        def _():
            pltpu.async_copy(in_hbm, buf, s0).wait()

            iota = lax.broadcasted_iota(jnp.int32, (_L,), 0)

            for g in range(tpad // _L):
                lt = iota + (g * _L)
                if (g + 1) * _L > T:
                    lt = jnp.minimum(lt, T - 1)
                fi0 = plsc.bitcast(plsc.load_gather(buf, [lt * 3 + OF]), jnp.int32)
                fi1 = plsc.bitcast(plsc.load_gather(buf, [lt * 3 + (OF + 1)]), jnp.int32)
                fi2 = plsc.bitcast(plsc.load_gather(buf, [lt * 3 + (OF + 2)]), jnp.int32)
                uf0 = plsc.bitcast(plsc.load_gather(buf, [lt * 3 + OG]), jnp.int32)
                uf1 = plsc.bitcast(plsc.load_gather(buf, [lt * 3 + (OG + 1)]), jnp.int32)
                uf2 = plsc.bitcast(plsc.load_gather(buf, [lt * 3 + (OG + 2)]), jnp.int32)

                ax = plsc.load_gather(buf, [fi0 * 3 + OV])
                ay = plsc.load_gather(buf, [fi0 * 3 + (OV + 1)])
                az = plsc.load_gather(buf, [fi0 * 3 + (OV + 2)])
                bx = plsc.load_gather(buf, [fi1 * 3 + OV])
                by = plsc.load_gather(buf, [fi1 * 3 + (OV + 1)])
                bz = plsc.load_gather(buf, [fi1 * 3 + (OV + 2)])
                cx = plsc.load_gather(buf, [fi2 * 3 + OV])
                cy = plsc.load_gather(buf, [fi2 * 3 + (OV + 1)])
                cz = plsc.load_gather(buf, [fi2 * 3 + (OV + 2)])
                ua = plsc.load_gather(buf, [uf0 * 2 + OU])
                va = plsc.load_gather(buf, [uf0 * 2 + (OU + 1)])
                ub = plsc.load_gather(buf, [uf1 * 2 + OU])
                vb = plsc.load_gather(buf, [uf1 * 2 + (OU + 1)])
                uc = plsc.load_gather(buf, [uf2 * 2 + OU])
                vc = plsc.load_gather(buf, [uf2 * 2 + (OU + 1)])

                cross = (bx - ax) * (cy - ay) - (by - ay) * (cx - ax)
                w = 0.5 * cross
                valid = (cross > 0.0) & (w >= 1e-9)
                wsafe = jnp.where(w == 0.0, 1.0, w)
                h = 0.5 / wsafe

                def edge(qx, qy, rx, ry):
                    return ((qx * ry - qy * rx) * h,
                            (qy - ry) * h,
                            (rx - qx) * h)

                w1c0, w1cx, w1cy = edge(bx, by, cx, cy)
                w2c0, w2cx, w2cy = edge(cx, cy, ax, ay)
                a0c0, a0cx, a0cy = edge(ax, ay, bx, by)

                inf = jnp.float32(jnp.inf)
                xmin = jnp.where(valid, jnp.minimum(jnp.minimum(ax, bx), cx), inf)
                xmax = jnp.where(valid, jnp.maximum(jnp.maximum(ax, bx), cx), -inf)
                ymin = jnp.minimum(jnp.minimum(ay, by), cy)
                ymax = jnp.maximum(jnp.maximum(ay, by), cy)

                zia = 1.0 / az
                zib = 1.0 / bz
                zic = 1.0 / cz
                rows = [
                    xmin, xmax, ymin, ymax,
                    w1c0, w1cx, w1cy,
                    w2c0, w2cx, w2cy,
                    a0c0, a0cx, a0cy,
                    ua * zia, ub * zib, uc * zic,
                    va * zia, vb * zib, vc * zic,
                    zia, zib, zic,
                ]
                lanes = iota + (g * _L)
                for k, val in enumerate(rows):
                    plsc.store_scatter(tab, [lanes * 32 + k], val)

            pbase = wid * (2 * _L) + OP
            px = plsc.load_gather(buf, [iota * 2 + pbase])
            py = plsc.load_gather(buf, [iota * 2 + (pbase + 1)])
            px = px / (_SIZE - 1) * 2.0 - 1.0
            py = (_SIZE - py) / (_SIZE - 1) * 2.0 - 1.0

            def body(t, carry):
                bs, bu, bv = carry
                ca = tab[pl.ds(t * 32, _L)]
                cb = tab[pl.ds(t * 32 + _L, _L)]
                inb = ((px >= ca[0]) & (px <= ca[1])
                       & (py >= ca[2]) & (py <= ca[3]))
                w1 = ca[4] + ca[5] * px + ca[6] * py
                w2 = ca[7] + ca[8] * px + ca[9] * py
                a0 = ca[10] + ca[11] * px + ca[12] * py
                w3 = 1.0 - w1 - w2
                zi = w1 * cb[3] + w2 * cb[4] + w3 * cb[5]
                ptz = 1.0 / zi
                uu = (w1 * ca[13] + w2 * ca[14] + w3 * ca[15]) * ptz
                vv = (w1 * cb[0] + w2 * cb[1] + w3 * cb[2]) * ptz
                upd = (inb & (w1 >= 0.0) & (w2 >= 0.0) & (a0 >= 0.0)
                       & (ptz > bs))
                return (jnp.where(upd, ptz, bs),
                        jnp.where(upd, uu, bu),
                        jnp.where(upd, vv, bv))

            bs, bu, bv = lax.fori_loop(
                0, T, body,
                (jnp.full((_L,), -jnp.inf, jnp.float32),
                 jnp.zeros((_L,), jnp.float32),
                 jnp.zeros((_L,), jnp.float32)))

            plsc.store_scatter(obuf, [iota * 3], bu)
            plsc.store_scatter(obuf, [iota * 3 + 1], bv)
            plsc.store_scatter(obuf, [iota * 3 + 2], bs)
            pltpu.sync_copy(obuf, out_hbm.at[pl.ds(wid * 3 * _L, 3 * _L)])

    return project


def kernel(vertices, points, faces, uv, uvfaces):
    T = faces.shape[0]
    P = points.shape[0]
    NV = vertices.shape[0]
    NU = uv.shape[0]

    fbits = lax.bitcast_convert_type(faces.astype(jnp.int32), jnp.float32)
    ubits = lax.bitcast_convert_type(uvfaces.astype(jnp.int32), jnp.float32)
    packed = jnp.concatenate([
        vertices.reshape(-1),
        uv.reshape(-1),
        fbits.reshape(-1),
        ubits.reshape(-1),
        points.reshape(-1),
    ])

    out = _make_project(T, P, NV, NU)(packed)
    return out.reshape(P, 3)


# speedup vs baseline: 1.0726x; 1.0119x over previous
import functools

import jax
import jax.numpy as jnp
from jax import lax
from jax.experimental import pallas as pl
from jax.experimental.pallas import tpu as pltpu
from jax.experimental.pallas import tpu_sc as plsc

_SIZE = 512
_L = 16
_NC = 2
_NS = 16


@functools.lru_cache(maxsize=None)
def _make_project(T, P, NV, NU):
    tpad = -(-T // _L) * _L
    nchunk = P // _L
    OV = 0
    OU = 3 * NV
    OF = OU + 2 * NU
    OG = OF + 3 * T
    OP = OG + 3 * T
    NIN = OP + 2 * P

    mesh = plsc.VectorSubcoreMesh(
        core_axis_name="c", subcore_axis_name="s", num_cores=_NC, num_subcores=_NS
    )

    @functools.partial(
        pl.kernel,
        out_type=jax.ShapeDtypeStruct((P * 3,), jnp.float32),
        mesh=mesh,
        compiler_params=pltpu.CompilerParams(needs_layout_passes=False),
        scratch_types=[
            pltpu.VMEM((NIN,), jnp.float32),
            pltpu.VMEM((tpad * 32,), jnp.float32),
            pltpu.VMEM((3 * _L,), jnp.float32),
            pltpu.SemaphoreType.DMA,
        ],
    )
    def project(in_hbm, out_hbm, buf, tab, obuf, s0):
        wid = lax.axis_index("s") * _NC + lax.axis_index("c")

        @pl.when(wid < nchunk)
        def _():
            pltpu.async_copy(in_hbm, buf, s0).wait()

            iota = lax.broadcasted_iota(jnp.int32, (_L,), 0)

            def phase1(g, carry):
                lanes = iota + g * _L
                lt = jnp.minimum(lanes, T - 1)
                fi0 = plsc.bitcast(plsc.load_gather(buf, [lt * 3 + OF]), jnp.int32)
                fi1 = plsc.bitcast(plsc.load_gather(buf, [lt * 3 + (OF + 1)]), jnp.int32)
                fi2 = plsc.bitcast(plsc.load_gather(buf, [lt * 3 + (OF + 2)]), jnp.int32)
                uf0 = plsc.bitcast(plsc.load_gather(buf, [lt * 3 + OG]), jnp.int32)
                uf1 = plsc.bitcast(plsc.load_gather(buf, [lt * 3 + (OG + 1)]), jnp.int32)
                uf2 = plsc.bitcast(plsc.load_gather(buf, [lt * 3 + (OG + 2)]), jnp.int32)

                ax = plsc.load_gather(buf, [fi0 * 3 + OV])
                ay = plsc.load_gather(buf, [fi0 * 3 + (OV + 1)])
                az = plsc.load_gather(buf, [fi0 * 3 + (OV + 2)])
                bx = plsc.load_gather(buf, [fi1 * 3 + OV])
                by = plsc.load_gather(buf, [fi1 * 3 + (OV + 1)])
                bz = plsc.load_gather(buf, [fi1 * 3 + (OV + 2)])
                cx = plsc.load_gather(buf, [fi2 * 3 + OV])
                cy = plsc.load_gather(buf, [fi2 * 3 + (OV + 1)])
                cz = plsc.load_gather(buf, [fi2 * 3 + (OV + 2)])
                ua = plsc.load_gather(buf, [uf0 * 2 + OU])
                va = plsc.load_gather(buf, [uf0 * 2 + (OU + 1)])
                ub = plsc.load_gather(buf, [uf1 * 2 + OU])
                vb = plsc.load_gather(buf, [uf1 * 2 + (OU + 1)])
                uc = plsc.load_gather(buf, [uf2 * 2 + OU])
                vc = plsc.load_gather(buf, [uf2 * 2 + (OU + 1)])

                cross = (bx - ax) * (cy - ay) - (by - ay) * (cx - ax)
                w = 0.5 * cross
                valid = (cross > 0.0) & (w >= 1e-9)
                wsafe = jnp.where(w == 0.0, 1.0, w)
                h = 0.5 / wsafe

                def edge(qx, qy, rx, ry):
                    return ((qx * ry - qy * rx) * h,
                            (qy - ry) * h,
                            (rx - qx) * h)

                w1c0, w1cx, w1cy = edge(bx, by, cx, cy)
                w2c0, w2cx, w2cy = edge(cx, cy, ax, ay)
                a0c0, a0cx, a0cy = edge(ax, ay, bx, by)

                inf = jnp.float32(jnp.inf)
                xmin = jnp.where(valid, jnp.minimum(jnp.minimum(ax, bx), cx), inf)
                xmax = jnp.where(valid, jnp.maximum(jnp.maximum(ax, bx), cx), -inf)
                ymin = jnp.minimum(jnp.minimum(ay, by), cy)
                ymax = jnp.maximum(jnp.maximum(ay, by), cy)

                zia = 1.0 / az
                zib = 1.0 / bz
                zic = 1.0 / cz
                rows = [
                    xmin, xmax, ymin, ymax,
                    w1c0, w1cx, w1cy,
                    w2c0, w2cx, w2cy,
                    a0c0, a0cx, a0cy,
                    ua * zia, ub * zib, uc * zic,
                    va * zia, vb * zib, vc * zic,
                    zia, zib, zic,
                ]
                for k, val in enumerate(rows):
                    plsc.store_scatter(tab, [lanes * 32 + k], val)
                return carry

            lax.fori_loop(0, tpad // _L, phase1, 0)

            pbase = wid * (2 * _L) + OP
            px = plsc.load_gather(buf, [iota * 2 + pbase])
            py = plsc.load_gather(buf, [iota * 2 + (pbase + 1)])
            px = px / (_SIZE - 1) * 2.0 - 1.0
            py = (_SIZE - py) / (_SIZE - 1) * 2.0 - 1.0

            def body(t, carry):
                bs, bu, bv = carry
                ca = tab[pl.ds(t * 32, _L)]
                cb = tab[pl.ds(t * 32 + _L, _L)]
                inb = ((px >= ca[0]) & (px <= ca[1])
                       & (py >= ca[2]) & (py <= ca[3]))
                w1 = ca[4] + ca[5] * px + ca[6] * py
                w2 = ca[7] + ca[8] * px + ca[9] * py
                a0 = ca[10] + ca[11] * px + ca[12] * py
                w3 = 1.0 - w1 - w2
                zi = w1 * cb[3] + w2 * cb[4] + w3 * cb[5]
                ptz = 1.0 / zi
                uu = (w1 * ca[13] + w2 * ca[14] + w3 * ca[15]) * ptz
                vv = (w1 * cb[0] + w2 * cb[1] + w3 * cb[2]) * ptz
                upd = (inb & (w1 >= 0.0) & (w2 >= 0.0) & (a0 >= 0.0)
                       & (ptz > bs))
                return (jnp.where(upd, ptz, bs),
                        jnp.where(upd, uu, bu),
                        jnp.where(upd, vv, bv))

            bs, bu, bv = lax.fori_loop(
                0, T, body,
                (jnp.full((_L,), -jnp.inf, jnp.float32),
                 jnp.zeros((_L,), jnp.float32),
                 jnp.zeros((_L,), jnp.float32)))

            plsc.store_scatter(obuf, [iota * 3], bu)
            plsc.store_scatter(obuf, [iota * 3 + 1], bv)
            plsc.store_scatter(obuf, [iota * 3 + 2], bs)
            pltpu.sync_copy(obuf, out_hbm.at[pl.ds(wid * 3 * _L, 3 * _L)])

    return project


def kernel(vertices, points, faces, uv, uvfaces):
    T = faces.shape[0]
    P = points.shape[0]
    NV = vertices.shape[0]
    NU = uv.shape[0]

    fbits = lax.bitcast_convert_type(faces.astype(jnp.int32), jnp.float32)
    ubits = lax.bitcast_convert_type(uvfaces.astype(jnp.int32), jnp.float32)
    packed = jnp.concatenate([
        vertices.reshape(-1),
        uv.reshape(-1),
        fbits.reshape(-1),
        ubits.reshape(-1),
        points.reshape(-1),
    ])

    out = _make_project(T, P, NV, NU)(packed)
    return out.reshape(P, 3)


# speedup vs baseline: 1.1097x; 1.0346x over previous
import functools

import jax
import jax.numpy as jnp
from jax import lax
from jax.experimental import pallas as pl
from jax.experimental.pallas import tpu as pltpu
from jax.experimental.pallas import tpu_sc as plsc

_SIZE = 512
_L = 16
_NC = 2
_NS = 16


@functools.lru_cache(maxsize=None)
def _make_project(T, P, NV, NU):
    tpad = -(-T // _L) * _L
    nchunk = P // _L
    t8 = -(-T // 8) * 8
    RV = 0
    RU = RV + NV
    RF = RU + NU
    RG = RF + t8
    RP = RG + t8
    NIN = 3 * (RP + P)

    mesh = plsc.VectorSubcoreMesh(
        core_axis_name="c", subcore_axis_name="s", num_cores=_NC, num_subcores=_NS
    )

    @functools.partial(
        pl.kernel,
        out_type=jax.ShapeDtypeStruct((P * 3,), jnp.float32),
        mesh=mesh,
        compiler_params=pltpu.CompilerParams(needs_layout_passes=False),
        scratch_types=[
            pltpu.VMEM((NIN,), jnp.float32),
            pltpu.VMEM((tpad * 32,), jnp.float32),
            pltpu.VMEM((3 * _L,), jnp.float32),
            pltpu.SemaphoreType.DMA,
        ],
    )
    def project(in_hbm, out_hbm, buf, tab, obuf, s0):
        wid = lax.axis_index("s") * _NC + lax.axis_index("c")

        @pl.when(wid < nchunk)
        def _():
            pltpu.async_copy(in_hbm, buf, s0).wait()

            iota = lax.broadcasted_iota(jnp.int32, (_L,), 0)

            def phase1(g, carry):
                lanes = iota + g * _L
                lt = jnp.minimum(lanes, T - 1)
                ft = (lt + RF) * 3
                gt = (lt + RG) * 3
                fi0 = plsc.bitcast(plsc.load_gather(buf, [ft]), jnp.int32)
                fi1 = plsc.bitcast(plsc.load_gather(buf, [ft + 1]), jnp.int32)
                fi2 = plsc.bitcast(plsc.load_gather(buf, [ft + 2]), jnp.int32)
                uf0 = plsc.bitcast(plsc.load_gather(buf, [gt]), jnp.int32)
                uf1 = plsc.bitcast(plsc.load_gather(buf, [gt + 1]), jnp.int32)
                uf2 = plsc.bitcast(plsc.load_gather(buf, [gt + 2]), jnp.int32)

                ax = plsc.load_gather(buf, [fi0 * 3])
                ay = plsc.load_gather(buf, [fi0 * 3 + 1])
                az = plsc.load_gather(buf, [fi0 * 3 + 2])
                bx = plsc.load_gather(buf, [fi1 * 3])
                by = plsc.load_gather(buf, [fi1 * 3 + 1])
                bz = plsc.load_gather(buf, [fi1 * 3 + 2])
                cx = plsc.load_gather(buf, [fi2 * 3])
                cy = plsc.load_gather(buf, [fi2 * 3 + 1])
                cz = plsc.load_gather(buf, [fi2 * 3 + 2])
                ua = plsc.load_gather(buf, [(uf0 + RU) * 3])
                va = plsc.load_gather(buf, [(uf0 + RU) * 3 + 1])
                ub = plsc.load_gather(buf, [(uf1 + RU) * 3])
                vb = plsc.load_gather(buf, [(uf1 + RU) * 3 + 1])
                uc = plsc.load_gather(buf, [(uf2 + RU) * 3])
                vc = plsc.load_gather(buf, [(uf2 + RU) * 3 + 1])

                cross = (bx - ax) * (cy - ay) - (by - ay) * (cx - ax)
                w = 0.5 * cross
                valid = (cross > 0.0) & (w >= 1e-9)
                wsafe = jnp.where(w == 0.0, 1.0, w)
                h = 0.5 / wsafe

                def edge(qx, qy, rx, ry):
                    return ((qx * ry - qy * rx) * h,
                            (qy - ry) * h,
                            (rx - qx) * h)

                w1c0, w1cx, w1cy = edge(bx, by, cx, cy)
                w2c0, w2cx, w2cy = edge(cx, cy, ax, ay)
                a0c0, a0cx, a0cy = edge(ax, ay, bx, by)

                inf = jnp.float32(jnp.inf)
                xmin = jnp.where(valid, jnp.minimum(jnp.minimum(ax, bx), cx), inf)
                xmax = jnp.where(valid, jnp.maximum(jnp.maximum(ax, bx), cx), -inf)
                ymin = jnp.minimum(jnp.minimum(ay, by), cy)
                ymax = jnp.maximum(jnp.maximum(ay, by), cy)

                zia = 1.0 / az
                zib = 1.0 / bz
                zic = 1.0 / cz
                rows = [
                    xmin, xmax, ymin, ymax,
                    w1c0, w1cx, w1cy,
                    w2c0, w2cx, w2cy,
                    a0c0, a0cx, a0cy,
                    ua * zia, ub * zib, uc * zic,
                    va * zia, vb * zib, vc * zic,
                    zia, zib, zic,
                ]
                for k, val in enumerate(rows):
                    plsc.store_scatter(tab, [lanes * 32 + k], val)
                return carry

            lax.fori_loop(0, tpad // _L, phase1, 0)

            prow = (iota + (wid * _L + RP)) * 3
            px = plsc.load_gather(buf, [prow])
            py = plsc.load_gather(buf, [prow + 1])
            px = px / (_SIZE - 1) * 2.0 - 1.0
            py = (_SIZE - py) / (_SIZE - 1) * 2.0 - 1.0

            def body(t, carry):
                bs, bu, bv = carry
                ca = tab[pl.ds(t * 32, _L)]
                cb = tab[pl.ds(t * 32 + _L, _L)]
                inb = ((px >= ca[0]) & (px <= ca[1])
                       & (py >= ca[2]) & (py <= ca[3]))
                w1 = ca[4] + ca[5] * px + ca[6] * py
                w2 = ca[7] + ca[8] * px + ca[9] * py
                a0 = ca[10] + ca[11] * px + ca[12] * py
                w3 = 1.0 - w1 - w2
                zi = w1 * cb[3] + w2 * cb[4] + w3 * cb[5]
                ptz = 1.0 / zi
                uu = (w1 * ca[13] + w2 * ca[14] + w3 * ca[15]) * ptz
                vv = (w1 * cb[0] + w2 * cb[1] + w3 * cb[2]) * ptz
                upd = (inb & (w1 >= 0.0) & (w2 >= 0.0) & (a0 >= 0.0)
                       & (ptz > bs))
                return (jnp.where(upd, ptz, bs),
                        jnp.where(upd, uu, bu),
                        jnp.where(upd, vv, bv))

            bs, bu, bv = lax.fori_loop(
                0, T, body,
                (jnp.full((_L,), -jnp.inf, jnp.float32),
                 jnp.zeros((_L,), jnp.float32),
                 jnp.zeros((_L,), jnp.float32)))

            plsc.store_scatter(obuf, [iota * 3], bu)
            plsc.store_scatter(obuf, [iota * 3 + 1], bv)
            plsc.store_scatter(obuf, [iota * 3 + 2], bs)
            pltpu.sync_copy(obuf, out_hbm.at[pl.ds(wid * 3 * _L, 3 * _L)])

    return project


def kernel(vertices, points, faces, uv, uvfaces):
    T = faces.shape[0]
    P = points.shape[0]
    NV = vertices.shape[0]
    NU = uv.shape[0]

    t8 = -(-T // 8) * 8
    fbits = lax.bitcast_convert_type(faces.astype(jnp.int32), jnp.float32)
    ubits = lax.bitcast_convert_type(uvfaces.astype(jnp.int32), jnp.float32)
    packed = jnp.concatenate([
        vertices,
        jnp.pad(uv, ((0, 0), (0, 1))),
        jnp.pad(fbits, ((0, t8 - T), (0, 0))),
        jnp.pad(ubits, ((0, t8 - T), (0, 0))),
        jnp.pad(points, ((0, 0), (0, 1))),
    ], axis=0).reshape(-1)

    out = _make_project(T, P, NV, NU)(packed)
    return out.reshape(P, 3)
